# baseline (speedup 1.0000x reference)
import functools

import jax
import jax.numpy as jnp
import numpy as np
from jax import lax
from jax.experimental import pallas as pl
from jax.experimental.pallas import tpu as pltpu
from jax.experimental.pallas import tpu_sc as plsc

_EMBED = 256
_HEADS = 8
_LEVELS = 4
_POINTS = 4
_HD = _EMBED // _HEADS
_NW = 32
_QB = 32


def _matmul_bias_kernel(x_ref, w_ref, b_ref, o_ref):
    o_ref[...] = (
        jnp.dot(x_ref[...], w_ref[...], preferred_element_type=jnp.float32)
        + b_ref[0:1, :]
    )


def _matmul_bias(x, w, b, block_rows, out_rows=None):
    rows = x.shape[0]
    assert rows % block_rows == 0, (rows, block_rows)
    out_rows = rows if out_rows is None else out_rows
    k, n = w.shape
    b2 = jnp.broadcast_to(b.reshape(1, n), (8, n))
    return pl.pallas_call(
        _matmul_bias_kernel,
        grid=(rows // block_rows,),
        in_specs=[
            pl.BlockSpec((block_rows, k), lambda i: (i, 0)),
            pl.BlockSpec((k, n), lambda i: (0, 0)),
            pl.BlockSpec((8, n), lambda i: (0, 0)),
        ],
        out_specs=pl.BlockSpec((block_rows, n), lambda i: (i, 0)),
        out_shape=jax.ShapeDtypeStruct((out_rows, n), jnp.float32),
    )(x, w, b2)


def _proj_kernel(nblks, x0_ref, x1_ref, x2_ref, x3_ref, we_ref, wo_ref,
                 be_ref, bo_ref, o_ref):
    i = pl.program_id(0)
    bnd = [0] + list(nblks)

    for l, x_ref in enumerate((x0_ref, x1_ref, x2_ref, x3_ref)):
        @pl.when((i >= bnd[l]) & (i < bnd[l + 1]))
        def _(x_ref=x_ref):
            x = x_ref[...]
            re = (jnp.dot(x, we_ref[...], preferred_element_type=jnp.float32)
                  + be_ref[0:1, :])
            ro = (jnp.dot(x, wo_ref[...], preferred_element_type=jnp.float32)
                  + bo_ref[0:1, :])

            def rne16(f):
                u = lax.bitcast_convert_type(f, jnp.uint32)
                return (u + jnp.uint32(0x7FFF)
                        + ((u >> jnp.uint32(16)) & jnp.uint32(1))
                        ) >> jnp.uint32(16)

            o_ref[...] = lax.bitcast_convert_type(
                rne16(re) | (rne16(ro) << jnp.uint32(16)), jnp.int32)


def _project_values(vals2d, w, b, block_rows):
    we = w[:, 0::2]
    wo = w[:, 1::2]
    n = we.shape[1]
    be2 = jnp.broadcast_to(b[0::2].reshape(1, n), (8, n))
    bo2 = jnp.broadcast_to(b[1::2].reshape(1, n), (8, n))
    sizes = [v.shape[0] for v in vals2d]
    nblks = []
    acc = 0
    for s in sizes:
        assert s % block_rows == 0
        acc += s // block_rows
        nblks.append(acc)
    starts = [e - s // block_rows for e, s in zip(nblks, sizes)]

    def mk_map(start, nb):
        return lambda i: (jnp.clip(i - start, 0, nb - 1), 0)

    in_specs = [
        pl.BlockSpec((block_rows, _EMBED), mk_map(st, sz // block_rows))
        for st, sz in zip(starts, sizes)
    ] + [
        pl.BlockSpec((_EMBED, n), lambda i: (0, 0)),
        pl.BlockSpec((_EMBED, n), lambda i: (0, 0)),
        pl.BlockSpec((8, n), lambda i: (0, 0)),
        pl.BlockSpec((8, n), lambda i: (0, 0)),
    ]
    total = sum(sizes)
    return pl.pallas_call(
        functools.partial(_proj_kernel, nblks),
        grid=(total // block_rows,),
        in_specs=in_specs,
        out_specs=pl.BlockSpec((block_rows, n), lambda i: (i, 0)),
        out_shape=jax.ShapeDtypeStruct((total, n), jnp.int32),
    )(*vals2d, we, wo, be2, bo2)


def _idxw_math(q, refi, refj, bid, wsi, wsj, bsi, bsj, wa, ba, gmat, scale,
               baseh, hw, maxi):
    offi = jnp.dot(q, wsi, preferred_element_type=jnp.float32) + bsi
    offj = jnp.dot(q, wsj, preferred_element_type=jnp.float32) + bsj
    logits = jnp.dot(q, wa, preferred_element_type=jnp.float32) + ba
    m = jnp.max(logits, axis=-1, keepdims=True)
    e = jnp.exp(logits - m)
    s = jnp.dot(e, gmat, preferred_element_type=jnp.float32)
    attn = e / s

    loci = (refi + offi) * scale
    locj = (refj + offj) * scale
    sci = jnp.maximum(loci - 0.5, 0.0)
    scj = jnp.maximum(locj - 0.5, 0.0)
    fli = jnp.floor(sci)
    flj = jnp.floor(scj)
    i0 = fli.astype(jnp.int32)
    j0 = flj.astype(jnp.int32)
    fri = sci - fli
    frj = scj - flj
    i0c = jnp.minimum(i0, maxi)
    i1c = jnp.minimum(i0 + 1, maxi)
    j0c = jnp.minimum(j0, maxi)
    j1c = jnp.minimum(j0 + 1, maxi)

    hw8 = hw * _HEADS
    ai0 = baseh + (bid * hw + i0c) * hw8
    ai1 = baseh + (bid * hw + i1c) * hw8
    bj0 = j0c * _HEADS
    bj1 = j1c * _HEADS
    idx = jnp.concatenate(
        [ai0 + bj0, ai0 + bj1, ai1 + bj0, ai1 + bj1], axis=-1)
    wi0 = 1.0 - fri
    wj0 = 1.0 - frj
    w = jnp.concatenate(
        [wi0 * wj0 * attn, wi0 * frj * attn, fri * wj0 * attn,
         fri * frj * attn], axis=-1)
    return idx, w


def _idxw_kernel(nb, q_ref, rp_ref, offs_ref, wsi_ref, wsj_ref, bsi_ref,
                 bsj_ref, wa_ref, ba_ref, g_ref, scale_ref, baseh_ref, hw_ref,
                 maxi_ref, idx_ref, w_ref):
    blk = q_ref.shape[0]
    rows = (lax.broadcasted_iota(jnp.int32, (blk, 128), 0)
            + pl.program_id(0) * blk)
    bid = jnp.full((blk, 128), -1, jnp.int32)
    for k in range(nb):
        bid = bid + (rows >= offs_ref[0, k]).astype(jnp.int32)
    idx, w = _idxw_math(
        q_ref[...], rp_ref[:, 0:1], rp_ref[:, 1:2], bid, wsi_ref[...],
        wsj_ref[...], bsi_ref[0:1, :], bsj_ref[0:1, :], wa_ref[...],
        ba_ref[0:1, :], g_ref[...], scale_ref[0:1, :], baseh_ref[0:1, :],
        hw_ref[0:1, :], maxi_ref[0:1, :])
    idx_ref[...] = idx
    w_ref[...] = w


def _compute_idxw(qpad, rp_pad, offs, wsi, wsj, bsi, bsj, wa, ba, hw_consts,
                  base_consts, block_rows):
    npad = qpad.shape[0]
    nb = offs.shape[0]
    offs2 = jnp.broadcast_to(
        jnp.pad(offs, (0, 128 - nb)).reshape(1, 128), (8, 128))
    bsi2 = jnp.broadcast_to(bsi.reshape(1, 128), (8, 128))
    bsj2 = jnp.broadcast_to(bsj.reshape(1, 128), (8, 128))
    ba2 = jnp.broadcast_to(ba.reshape(1, 128), (8, 128))
    col = jnp.arange(128, dtype=jnp.int32)
    gmat = (col[:, None] // 16 == col[None, :] // 16).astype(jnp.float32)
    level = (col % 16) // 4
    hw_v = jnp.array(hw_consts, jnp.int32)[level]
    base_v = jnp.array(base_consts, jnp.int32)[level] + col // 16
    scale_v = hw_v.astype(jnp.float32) * (1.0 / float(hw_consts[3]))
    maxi_v = hw_v - 1
    scale2 = jnp.broadcast_to(scale_v.reshape(1, 128), (8, 128))
    baseh2 = jnp.broadcast_to(base_v.reshape(1, 128), (8, 128))
    hw2 = jnp.broadcast_to(hw_v.reshape(1, 128), (8, 128))
    maxi2 = jnp.broadcast_to(maxi_v.reshape(1, 128), (8, 128))
    cspec = pl.BlockSpec((8, 128), lambda i: (0, 0))
    return pl.pallas_call(
        functools.partial(_idxw_kernel, nb),
        grid=(npad // block_rows,),
        in_specs=[
            pl.BlockSpec((block_rows, _EMBED), lambda i: (i, 0)),
            pl.BlockSpec((block_rows, 2), lambda i: (i, 0)),
            cspec,
            pl.BlockSpec((_EMBED, 128), lambda i: (0, 0)),
            pl.BlockSpec((_EMBED, 128), lambda i: (0, 0)),
            cspec,
            cspec,
            pl.BlockSpec((_EMBED, 128), lambda i: (0, 0)),
            cspec,
            pl.BlockSpec((128, 128), lambda i: (0, 0)),
            cspec,
            cspec,
            cspec,
            cspec,
        ],
        out_specs=[
            pl.BlockSpec((block_rows, 512), lambda i: (i, 0)),
            pl.BlockSpec((block_rows, 512), lambda i: (i, 0)),
        ],
        out_shape=[
            jax.ShapeDtypeStruct((npad, 512), jnp.int32),
            jax.ShapeDtypeStruct((npad, 512), jnp.float32),
        ],
    )(qpad, rp_pad, offs2, wsi, wsj, bsi2, bsj2, wa, ba2, gmat, scale2,
      baseh2, hw2, maxi2)


def _make_sc_gather(npad, nrows):
    per_w = npad // _NW
    nblk = per_w // _QB
    mesh = plsc.VectorSubcoreMesh(core_axis_name="c", subcore_axis_name="s")

    @functools.partial(
        pl.kernel,
        out_type=jax.ShapeDtypeStruct((npad, _EMBED), jnp.float32),
        mesh=mesh,
        scratch_types=[
            pltpu.VMEM((2, _QB, 4, 128), jnp.int32),
            pltpu.VMEM((2, _QB, 512), jnp.float32),
            pltpu.VMEM((6, 4, 128, 16), jnp.int32),
            pltpu.VMEM((_QB, _EMBED), jnp.float32),
            pltpu.SemaphoreType.DMA,
            pltpu.SemaphoreType.DMA,
        ],
        compiler_params=pltpu.CompilerParams(use_tc_tiling_on_sc=False),
    )
    def sc_kernel(table_hbm, idx_hbm, w_hbm, out_hbm, idx_v, w_v, rows_v,
                  out_v, sem, semb):
        wid = lax.axis_index("s") * 2 + lax.axis_index("c")

        def start_blk(b):
            st = wid * per_w + b * _QB
            ib = lax.rem(b, 2)
            pltpu.async_copy(idx_hbm.at[pl.ds(st, _QB)], idx_v.at[ib], semb)
            pltpu.async_copy(w_hbm.at[pl.ds(st, _QB)], w_v.at[ib], semb)

        def wait_blk(b):
            st = wid * per_w + b * _QB
            ib = lax.rem(b, 2)
            pltpu.make_async_copy(idx_hbm.at[pl.ds(st, _QB)], idx_v.at[ib],
                                  semb).wait()
            pltpu.make_async_copy(w_hbm.at[pl.ds(st, _QB)], w_v.at[ib],
                                  semb).wait()

        start_blk(0)

        def blk_body(bi, carry):
            start = wid * per_w + bi * _QB
            ib = lax.rem(bi, 2)
            wait_blk(bi)

            @pl.when(bi + 1 < nblk)
            def _():
                start_blk(bi + 1)

            def issue(qi, buf):
                for c in range(4):
                    pltpu.async_copy(table_hbm.at[idx_v.at[ib, qi, c]],
                                     rows_v.at[buf, c], sem)

            for k in range(5):
                issue(k, k)

            def q_body(qi, c2):
                buf = lax.rem(qi, 6)

                @pl.when(qi + 5 < _QB)
                def _():
                    issue(qi + 5, lax.rem(qi + 5, 6))

                for c in range(4):
                    pltpu.make_async_copy(table_hbm.at[idx_v.at[ib, qi, c]],
                                          rows_v.at[buf, c], sem).wait()
                zero = jnp.zeros((16,), jnp.float32)
                accs = (zero,) * (2 * _HEADS)

                for c in range(4):
                    wvecs = [
                        w_v[ib, qi, pl.ds(c * 128 + h * 16, 16)]
                        for h in range(_HEADS)
                    ]

                    def t_body(t, accs, c=c, wvecs=wvecs):
                        accs = list(accs)
                        tvec = jnp.full((16, 1), t, jnp.int32)
                        dnums = lax.GatherDimensionNumbers(
                            offset_dims=(), collapsed_slice_dims=(0,),
                            start_index_map=(0,))
                        for h in range(_HEADS):
                            wb = lax.gather(
                                wvecs[h], tvec, dnums, (1,),
                                mode=lax.GatherScatterMode.PROMISE_IN_BOUNDS)
                            r = h * 16 + t
                            u = rows_v[buf, c, r, :]
                            r0 = lax.bitcast_convert_type(u << 16,
                                                          jnp.float32)
                            r1 = lax.bitcast_convert_type(
                                u & jnp.int32(-65536), jnp.float32)
                            accs[2 * h] = accs[2 * h] + wb * r0
                            accs[2 * h + 1] = accs[2 * h + 1] + wb * r1
                        return tuple(accs)

                    accs = lax.fori_loop(0, 16, t_body, accs)
                for h in range(_HEADS):
                    out_v[qi, pl.ds(h * 32, 16)] = accs[2 * h]
                    out_v[qi, pl.ds(h * 32 + 16, 16)] = accs[2 * h + 1]
                return c2

            lax.fori_loop(0, _QB, q_body, 0)
            pltpu.sync_copy(out_v, out_hbm.at[pl.ds(start, _QB)])
            return carry

        lax.fori_loop(0, nblk, blk_body, 0)

    return sc_kernel


def kernel(query, query_offsets, reference_points, value_l0, value_l1,
           value_l2, value_l3, W_sampling, b_sampling, W_attn, b_attn,
           W_value, b_value, W_out, b_out):
    values = [value_l0, value_l1, value_l2, value_l3]
    N = query.shape[0]
    B = value_l0.shape[0]

    vals2d = [v.reshape(-1, _EMBED) for v in values]
    P = sum(v.shape[0] for v in vals2d)
    table = _project_values(vals2d, W_value, b_value, block_rows=512)
    table = table.reshape(P * _HEADS, 16)

    hw_consts = [v.shape[1] for v in values]
    pix_prefix = []
    acc = 0
    for v in values:
        pix_prefix.append(acc)
        acc += B * v.shape[1] * v.shape[2]
    base_consts = [p * _HEADS for p in pix_prefix]

    npad = ((N + _NW * _QB - 1) // (_NW * _QB)) * (_NW * _QB)
    qpad = jnp.pad(query, ((0, npad - N), (0, 0)))
    rp_pad = jnp.pad(reference_points, ((0, npad - N), (0, 0)))

    wsi = W_sampling[:, 0::2]
    wsj = W_sampling[:, 1::2]
    bsi = b_sampling[0::2]
    bsj = b_sampling[1::2]
    idx, w = _compute_idxw(qpad, rp_pad, query_offsets, wsi, wsj, bsi, bsj,
                           W_attn, b_attn, hw_consts, base_consts,
                           block_rows=1024)

    idx3 = idx.reshape(npad, 4, 128)
    sc_gather = _make_sc_gather(npad, table.shape[0])
    sampled = sc_gather(table, idx3, w)

    j = np.arange(_HD)
    within = np.where(j < 16, 2 * j, 2 * (j - 16) + 1)
    perm = (np.arange(_EMBED) // _HD) * _HD + within[np.arange(_EMBED) % _HD]
    w_out_perm = W_out[jnp.asarray(perm)]
    return _matmul_bias(sampled, w_out_perm, b_out, block_rows=1024,
                        out_rows=N)


# speedup vs baseline: 1.0070x; 1.0070x over previous
import functools

import jax
import jax.numpy as jnp
import numpy as np
from jax import lax
from jax.experimental import pallas as pl
from jax.experimental.pallas import tpu as pltpu
from jax.experimental.pallas import tpu_sc as plsc

_EMBED = 256
_HEADS = 8
_LEVELS = 4
_POINTS = 4
_HD = _EMBED // _HEADS
_NW = 32
_QB = 32


def _matmul_bias_kernel(x_ref, w_ref, b_ref, o_ref):
    o_ref[...] = (
        jnp.dot(x_ref[...], w_ref[...], preferred_element_type=jnp.float32)
        + b_ref[0:1, :]
    )


def _matmul_bias(x, w, b, block_rows, out_rows=None):
    rows = x.shape[0]
    assert rows % block_rows == 0, (rows, block_rows)
    out_rows = rows if out_rows is None else out_rows
    k, n = w.shape
    b2 = jnp.broadcast_to(b.reshape(1, n), (8, n))
    return pl.pallas_call(
        _matmul_bias_kernel,
        grid=(rows // block_rows,),
        in_specs=[
            pl.BlockSpec((block_rows, k), lambda i: (i, 0)),
            pl.BlockSpec((k, n), lambda i: (0, 0)),
            pl.BlockSpec((8, n), lambda i: (0, 0)),
        ],
        out_specs=pl.BlockSpec((block_rows, n), lambda i: (i, 0)),
        out_shape=jax.ShapeDtypeStruct((out_rows, n), jnp.float32),
    )(x, w, b2)


def _proj_kernel(nblks, x0_ref, x1_ref, x2_ref, x3_ref, we_ref, wo_ref,
                 be_ref, bo_ref, o_ref):
    i = pl.program_id(0)
    bnd = [0] + list(nblks)

    for l, x_ref in enumerate((x0_ref, x1_ref, x2_ref, x3_ref)):
        @pl.when((i >= bnd[l]) & (i < bnd[l + 1]))
        def _(x_ref=x_ref):
            x = x_ref[...]
            re = (jnp.dot(x, we_ref[...], preferred_element_type=jnp.float32)
                  + be_ref[0:1, :])
            ro = (jnp.dot(x, wo_ref[...], preferred_element_type=jnp.float32)
                  + bo_ref[0:1, :])

            def rne16(f):
                u = lax.bitcast_convert_type(f, jnp.uint32)
                return (u + jnp.uint32(0x7FFF)
                        + ((u >> jnp.uint32(16)) & jnp.uint32(1))
                        ) >> jnp.uint32(16)

            o_ref[...] = lax.bitcast_convert_type(
                rne16(re) | (rne16(ro) << jnp.uint32(16)), jnp.int32)


def _project_values(vals2d, w, b, block_rows):
    we = w[:, 0::2]
    wo = w[:, 1::2]
    n = we.shape[1]
    be2 = jnp.broadcast_to(b[0::2].reshape(1, n), (8, n))
    bo2 = jnp.broadcast_to(b[1::2].reshape(1, n), (8, n))
    sizes = [v.shape[0] for v in vals2d]
    nblks = []
    acc = 0
    for s in sizes:
        assert s % block_rows == 0
        acc += s // block_rows
        nblks.append(acc)
    starts = [e - s // block_rows for e, s in zip(nblks, sizes)]

    def mk_map(start, nb):
        return lambda i: (jnp.clip(i - start, 0, nb - 1), 0)

    in_specs = [
        pl.BlockSpec((block_rows, _EMBED), mk_map(st, sz // block_rows))
        for st, sz in zip(starts, sizes)
    ] + [
        pl.BlockSpec((_EMBED, n), lambda i: (0, 0)),
        pl.BlockSpec((_EMBED, n), lambda i: (0, 0)),
        pl.BlockSpec((8, n), lambda i: (0, 0)),
        pl.BlockSpec((8, n), lambda i: (0, 0)),
    ]
    total = sum(sizes)
    return pl.pallas_call(
        functools.partial(_proj_kernel, nblks),
        grid=(total // block_rows,),
        in_specs=in_specs,
        out_specs=pl.BlockSpec((block_rows, n), lambda i: (i, 0)),
        out_shape=jax.ShapeDtypeStruct((total, n), jnp.int32),
    )(*vals2d, we, wo, be2, bo2)


def _idxw_math(q, refi, refj, bid, wsi, wsj, bsi, bsj, wa, ba, gmat, scale,
               baseh, hw, maxi):
    offi = jnp.dot(q, wsi, preferred_element_type=jnp.float32) + bsi
    offj = jnp.dot(q, wsj, preferred_element_type=jnp.float32) + bsj
    logits = jnp.dot(q, wa, preferred_element_type=jnp.float32) + ba
    m = jnp.max(logits, axis=-1, keepdims=True)
    e = jnp.exp(logits - m)
    s = jnp.dot(e, gmat, preferred_element_type=jnp.float32)
    attn = e / s

    loci = (refi + offi) * scale
    locj = (refj + offj) * scale
    sci = jnp.maximum(loci - 0.5, 0.0)
    scj = jnp.maximum(locj - 0.5, 0.0)
    fli = jnp.floor(sci)
    flj = jnp.floor(scj)
    i0 = fli.astype(jnp.int32)
    j0 = flj.astype(jnp.int32)
    fri = sci - fli
    frj = scj - flj
    i0c = jnp.minimum(i0, maxi)
    i1c = jnp.minimum(i0 + 1, maxi)
    j0c = jnp.minimum(j0, maxi)
    j1c = jnp.minimum(j0 + 1, maxi)

    hw8 = hw * _HEADS
    ai0 = baseh + (bid * hw + i0c) * hw8
    ai1 = baseh + (bid * hw + i1c) * hw8
    bj0 = j0c * _HEADS
    bj1 = j1c * _HEADS
    idx = jnp.concatenate(
        [ai0 + bj0, ai0 + bj1, ai1 + bj0, ai1 + bj1], axis=-1)
    wi0 = 1.0 - fri
    wj0 = 1.0 - frj
    w = jnp.concatenate(
        [wi0 * wj0 * attn, wi0 * frj * attn, fri * wj0 * attn,
         fri * frj * attn], axis=-1)
    return idx, w


def _idxw_kernel(nb, q_ref, rp_ref, offs_ref, wsi_ref, wsj_ref, bsi_ref,
                 bsj_ref, wa_ref, ba_ref, g_ref, scale_ref, baseh_ref, hw_ref,
                 maxi_ref, idx_ref, w_ref):
    blk = q_ref.shape[0]
    rows = (lax.broadcasted_iota(jnp.int32, (blk, 128), 0)
            + pl.program_id(0) * blk)
    bid = jnp.full((blk, 128), -1, jnp.int32)
    for k in range(nb):
        bid = bid + (rows >= offs_ref[0, k]).astype(jnp.int32)
    idx, w = _idxw_math(
        q_ref[...], rp_ref[:, 0:1], rp_ref[:, 1:2], bid, wsi_ref[...],
        wsj_ref[...], bsi_ref[0:1, :], bsj_ref[0:1, :], wa_ref[...],
        ba_ref[0:1, :], g_ref[...], scale_ref[0:1, :], baseh_ref[0:1, :],
        hw_ref[0:1, :], maxi_ref[0:1, :])
    idx_ref[...] = idx
    w_ref[...] = w


def _compute_idxw(qpad, rp_pad, offs, wsi, wsj, bsi, bsj, wa, ba, hw_consts,
                  base_consts, block_rows):
    npad = qpad.shape[0]
    nb = offs.shape[0]
    offs2 = jnp.broadcast_to(
        jnp.pad(offs, (0, 128 - nb)).reshape(1, 128), (8, 128))
    bsi2 = jnp.broadcast_to(bsi.reshape(1, 128), (8, 128))
    bsj2 = jnp.broadcast_to(bsj.reshape(1, 128), (8, 128))
    ba2 = jnp.broadcast_to(ba.reshape(1, 128), (8, 128))
    col = jnp.arange(128, dtype=jnp.int32)
    gmat = (col[:, None] // 16 == col[None, :] // 16).astype(jnp.float32)
    level = (col % 16) // 4
    hw_v = jnp.array(hw_consts, jnp.int32)[level]
    base_v = jnp.array(base_consts, jnp.int32)[level] + col // 16
    scale_v = hw_v.astype(jnp.float32) * (1.0 / float(hw_consts[3]))
    maxi_v = hw_v - 1
    scale2 = jnp.broadcast_to(scale_v.reshape(1, 128), (8, 128))
    baseh2 = jnp.broadcast_to(base_v.reshape(1, 128), (8, 128))
    hw2 = jnp.broadcast_to(hw_v.reshape(1, 128), (8, 128))
    maxi2 = jnp.broadcast_to(maxi_v.reshape(1, 128), (8, 128))
    cspec = pl.BlockSpec((8, 128), lambda i: (0, 0))
    return pl.pallas_call(
        functools.partial(_idxw_kernel, nb),
        grid=(npad // block_rows,),
        in_specs=[
            pl.BlockSpec((block_rows, _EMBED), lambda i: (i, 0)),
            pl.BlockSpec((block_rows, 2), lambda i: (i, 0)),
            cspec,
            pl.BlockSpec((_EMBED, 128), lambda i: (0, 0)),
            pl.BlockSpec((_EMBED, 128), lambda i: (0, 0)),
            cspec,
            cspec,
            pl.BlockSpec((_EMBED, 128), lambda i: (0, 0)),
            cspec,
            pl.BlockSpec((128, 128), lambda i: (0, 0)),
            cspec,
            cspec,
            cspec,
            cspec,
        ],
        out_specs=[
            pl.BlockSpec((block_rows, 512), lambda i: (i, 0)),
            pl.BlockSpec((block_rows, 512), lambda i: (i, 0)),
        ],
        out_shape=[
            jax.ShapeDtypeStruct((npad, 512), jnp.int32),
            jax.ShapeDtypeStruct((npad, 512), jnp.float32),
        ],
    )(qpad, rp_pad, offs2, wsi, wsj, bsi2, bsj2, wa, ba2, gmat, scale2,
      baseh2, hw2, maxi2)


def _make_sc_gather(npad, nrows):
    per_w = npad // _NW
    nblk = per_w // _QB
    mesh = plsc.VectorSubcoreMesh(core_axis_name="c", subcore_axis_name="s")

    @functools.partial(
        pl.kernel,
        out_type=jax.ShapeDtypeStruct((npad, _EMBED), jnp.float32),
        mesh=mesh,
        scratch_types=[
            pltpu.VMEM((2, _QB, 4, 128), jnp.int32),
            pltpu.VMEM((2, _QB, 512), jnp.float32),
            pltpu.VMEM((5, 4, 128, 16), jnp.int32),
            pltpu.VMEM((_QB, _EMBED), jnp.float32),
            pltpu.SemaphoreType.DMA,
            pltpu.SemaphoreType.DMA,
        ],
        compiler_params=pltpu.CompilerParams(use_tc_tiling_on_sc=False),
    )
    def sc_kernel(table_hbm, idx_hbm, w_hbm, out_hbm, idx_v, w_v, rows_v,
                  out_v, sem, semb):
        wid = lax.axis_index("s") * 2 + lax.axis_index("c")

        def start_blk(b):
            st = wid * per_w + b * _QB
            ib = lax.rem(b, 2)
            pltpu.async_copy(idx_hbm.at[pl.ds(st, _QB)], idx_v.at[ib], semb)
            pltpu.async_copy(w_hbm.at[pl.ds(st, _QB)], w_v.at[ib], semb)

        def wait_blk(b):
            st = wid * per_w + b * _QB
            ib = lax.rem(b, 2)
            pltpu.make_async_copy(idx_hbm.at[pl.ds(st, _QB)], idx_v.at[ib],
                                  semb).wait()
            pltpu.make_async_copy(w_hbm.at[pl.ds(st, _QB)], w_v.at[ib],
                                  semb).wait()

        start_blk(0)

        def blk_body(bi, carry):
            start = wid * per_w + bi * _QB
            ib = lax.rem(bi, 2)
            wait_blk(bi)

            @pl.when(bi + 1 < nblk)
            def _():
                start_blk(bi + 1)

            def issue(qi, buf):
                for c in range(4):
                    pltpu.async_copy(table_hbm.at[idx_v.at[ib, qi, c]],
                                     rows_v.at[buf, c], sem)

            for k in range(4):
                issue(k, k)

            def q_body(qi, c2):
                buf = lax.rem(qi, 5)

                @pl.when(qi + 4 < _QB)
                def _():
                    issue(qi + 4, lax.rem(qi + 4, 5))

                for c in range(4):
                    pltpu.make_async_copy(table_hbm.at[idx_v.at[ib, qi, c]],
                                          rows_v.at[buf, c], sem).wait()
                zero = jnp.zeros((16,), jnp.float32)
                accs = (zero,) * (2 * _HEADS)

                for c in range(4):
                    wvecs = [
                        w_v[ib, qi, pl.ds(c * 128 + h * 16, 16)]
                        for h in range(_HEADS)
                    ]

                    def t_body(t, accs, c=c, wvecs=wvecs):
                        accs = list(accs)
                        tvec = jnp.full((16, 1), t, jnp.int32)
                        dnums = lax.GatherDimensionNumbers(
                            offset_dims=(), collapsed_slice_dims=(0,),
                            start_index_map=(0,))
                        for h in range(_HEADS):
                            wb = lax.gather(
                                wvecs[h], tvec, dnums, (1,),
                                mode=lax.GatherScatterMode.PROMISE_IN_BOUNDS)
                            r = h * 16 + t
                            u = rows_v[buf, c, r, :]
                            r0 = lax.bitcast_convert_type(u << 16,
                                                          jnp.float32)
                            r1 = lax.bitcast_convert_type(
                                u & jnp.int32(-65536), jnp.float32)
                            accs[2 * h] = accs[2 * h] + wb * r0
                            accs[2 * h + 1] = accs[2 * h + 1] + wb * r1
                        return tuple(accs)

                    accs = lax.fori_loop(0, 16, t_body, accs)
                for h in range(_HEADS):
                    out_v[qi, pl.ds(h * 32, 16)] = accs[2 * h]
                    out_v[qi, pl.ds(h * 32 + 16, 16)] = accs[2 * h + 1]
                return c2

            lax.fori_loop(0, _QB, q_body, 0)
            pltpu.sync_copy(out_v, out_hbm.at[pl.ds(start, _QB)])
            return carry

        lax.fori_loop(0, nblk, blk_body, 0)

    return sc_kernel


def kernel(query, query_offsets, reference_points, value_l0, value_l1,
           value_l2, value_l3, W_sampling, b_sampling, W_attn, b_attn,
           W_value, b_value, W_out, b_out):
    values = [value_l0, value_l1, value_l2, value_l3]
    N = query.shape[0]
    B = value_l0.shape[0]

    vals2d = [v.reshape(-1, _EMBED) for v in values]
    P = sum(v.shape[0] for v in vals2d)
    table = _project_values(vals2d, W_value, b_value, block_rows=512)
    table = table.reshape(P * _HEADS, 16)

    hw_consts = [v.shape[1] for v in values]
    pix_prefix = []
    acc = 0
    for v in values:
        pix_prefix.append(acc)
        acc += B * v.shape[1] * v.shape[2]
    base_consts = [p * _HEADS for p in pix_prefix]

    npad = ((N + _NW * _QB - 1) // (_NW * _QB)) * (_NW * _QB)
    qpad = jnp.pad(query, ((0, npad - N), (0, 0)))
    rp_pad = jnp.pad(reference_points, ((0, npad - N), (0, 0)))

    wsi = W_sampling[:, 0::2]
    wsj = W_sampling[:, 1::2]
    bsi = b_sampling[0::2]
    bsj = b_sampling[1::2]
    idx, w = _compute_idxw(qpad, rp_pad, query_offsets, wsi, wsj, bsi, bsj,
                           W_attn, b_attn, hw_consts, base_consts,
                           block_rows=1024)

    idx3 = idx.reshape(npad, 4, 128)
    sc_gather = _make_sc_gather(npad, table.shape[0])
    sampled = sc_gather(table, idx3, w)

    j = np.arange(_HD)
    within = np.where(j < 16, 2 * j, 2 * (j - 16) + 1)
    perm = (np.arange(_EMBED) // _HD) * _HD + within[np.arange(_EMBED) % _HD]
    w_out_perm = W_out[jnp.asarray(perm)]
    return _matmul_bias(sampled, w_out_perm, b_out, block_rows=1024,
                        out_rows=N)


# speedup vs baseline: 1.0190x; 1.0120x over previous
import functools

import jax
import jax.numpy as jnp
import numpy as np
from jax import lax
from jax.experimental import pallas as pl
from jax.experimental.pallas import tpu as pltpu
from jax.experimental.pallas import tpu_sc as plsc

_EMBED = 256
_HEADS = 8
_LEVELS = 4
_POINTS = 4
_HD = _EMBED // _HEADS
_NW = 32
_QB = 32


def _matmul_bias_kernel(x_ref, w_ref, b_ref, o_ref):
    o_ref[...] = (
        jnp.dot(x_ref[...], w_ref[...], preferred_element_type=jnp.float32)
        + b_ref[0:1, :]
    )


def _matmul_bias(x, w, b, block_rows, out_rows=None):
    rows = x.shape[0]
    assert rows % block_rows == 0, (rows, block_rows)
    out_rows = rows if out_rows is None else out_rows
    k, n = w.shape
    b2 = jnp.broadcast_to(b.reshape(1, n), (8, n))
    return pl.pallas_call(
        _matmul_bias_kernel,
        grid=(rows // block_rows,),
        in_specs=[
            pl.BlockSpec((block_rows, k), lambda i: (i, 0)),
            pl.BlockSpec((k, n), lambda i: (0, 0)),
            pl.BlockSpec((8, n), lambda i: (0, 0)),
        ],
        out_specs=pl.BlockSpec((block_rows, n), lambda i: (i, 0)),
        out_shape=jax.ShapeDtypeStruct((out_rows, n), jnp.float32),
    )(x, w, b2)


def _proj_kernel(nblks, x0_ref, x1_ref, x2_ref, x3_ref, we_ref, wo_ref,
                 be_ref, bo_ref, o_ref):
    i = pl.program_id(0)
    bnd = [0] + list(nblks)

    for l, x_ref in enumerate((x0_ref, x1_ref, x2_ref, x3_ref)):
        @pl.when((i >= bnd[l]) & (i < bnd[l + 1]))
        def _(x_ref=x_ref):
            x = x_ref[...]
            re = (jnp.dot(x, we_ref[...], preferred_element_type=jnp.float32)
                  + be_ref[0:1, :])
            ro = (jnp.dot(x, wo_ref[...], preferred_element_type=jnp.float32)
                  + bo_ref[0:1, :])

            def rne16(f):
                u = lax.bitcast_convert_type(f, jnp.uint32)
                return (u + jnp.uint32(0x7FFF)
                        + ((u >> jnp.uint32(16)) & jnp.uint32(1))
                        ) >> jnp.uint32(16)

            o_ref[...] = lax.bitcast_convert_type(
                rne16(re) | (rne16(ro) << jnp.uint32(16)), jnp.int32)


def _project_values(vals2d, w, b, block_rows):
    we = w[:, 0::2]
    wo = w[:, 1::2]
    n = we.shape[1]
    be2 = jnp.broadcast_to(b[0::2].reshape(1, n), (8, n))
    bo2 = jnp.broadcast_to(b[1::2].reshape(1, n), (8, n))
    sizes = [v.shape[0] for v in vals2d]
    nblks = []
    acc = 0
    for s in sizes:
        assert s % block_rows == 0
        acc += s // block_rows
        nblks.append(acc)
    starts = [e - s // block_rows for e, s in zip(nblks, sizes)]

    def mk_map(start, nb):
        return lambda i: (jnp.clip(i - start, 0, nb - 1), 0)

    in_specs = [
        pl.BlockSpec((block_rows, _EMBED), mk_map(st, sz // block_rows))
        for st, sz in zip(starts, sizes)
    ] + [
        pl.BlockSpec((_EMBED, n), lambda i: (0, 0)),
        pl.BlockSpec((_EMBED, n), lambda i: (0, 0)),
        pl.BlockSpec((8, n), lambda i: (0, 0)),
        pl.BlockSpec((8, n), lambda i: (0, 0)),
    ]
    total = sum(sizes)
    return pl.pallas_call(
        functools.partial(_proj_kernel, nblks),
        grid=(total // block_rows,),
        in_specs=in_specs,
        out_specs=pl.BlockSpec((block_rows, n), lambda i: (i, 0)),
        out_shape=jax.ShapeDtypeStruct((total, n), jnp.int32),
    )(*vals2d, we, wo, be2, bo2)


def _idxw_math(q, refi, refj, bid, wsi, wsj, bsi, bsj, wa, ba, gmat, scale,
               baseh, hw, maxi):
    offi = jnp.dot(q, wsi, preferred_element_type=jnp.float32) + bsi
    offj = jnp.dot(q, wsj, preferred_element_type=jnp.float32) + bsj
    logits = jnp.dot(q, wa, preferred_element_type=jnp.float32) + ba
    m = jnp.max(logits, axis=-1, keepdims=True)
    e = jnp.exp(logits - m)
    s = jnp.dot(e, gmat, preferred_element_type=jnp.float32)
    attn = e / s

    loci = (refi + offi) * scale
    locj = (refj + offj) * scale
    sci = jnp.maximum(loci - 0.5, 0.0)
    scj = jnp.maximum(locj - 0.5, 0.0)
    fli = jnp.floor(sci)
    flj = jnp.floor(scj)
    i0 = fli.astype(jnp.int32)
    j0 = flj.astype(jnp.int32)
    fri = sci - fli
    frj = scj - flj
    i0c = jnp.minimum(i0, maxi)
    i1c = jnp.minimum(i0 + 1, maxi)
    j0c = jnp.minimum(j0, maxi)
    j1c = jnp.minimum(j0 + 1, maxi)

    hw8 = hw * _HEADS
    ai0 = baseh + (bid * hw + i0c) * hw8
    ai1 = baseh + (bid * hw + i1c) * hw8
    bj0 = j0c * _HEADS
    bj1 = j1c * _HEADS
    idx = jnp.concatenate(
        [ai0 + bj0, ai0 + bj1, ai1 + bj0, ai1 + bj1], axis=-1)
    wi0 = 1.0 - fri
    wj0 = 1.0 - frj
    w = jnp.concatenate(
        [wi0 * wj0 * attn, wi0 * frj * attn, fri * wj0 * attn,
         fri * frj * attn], axis=-1)
    return idx, w


def _idxw_kernel(nb, q_ref, rp_ref, offs_ref, wsi_ref, wsj_ref, bsi_ref,
                 bsj_ref, wa_ref, ba_ref, g_ref, scale_ref, baseh_ref, hw_ref,
                 maxi_ref, idx_ref, w_ref):
    blk = q_ref.shape[0]
    rows = (lax.broadcasted_iota(jnp.int32, (blk, 128), 0)
            + pl.program_id(0) * blk)
    bid = jnp.full((blk, 128), -1, jnp.int32)
    for k in range(nb):
        bid = bid + (rows >= offs_ref[0, k]).astype(jnp.int32)
    idx, w = _idxw_math(
        q_ref[...], rp_ref[:, 0:1], rp_ref[:, 1:2], bid, wsi_ref[...],
        wsj_ref[...], bsi_ref[0:1, :], bsj_ref[0:1, :], wa_ref[...],
        ba_ref[0:1, :], g_ref[...], scale_ref[0:1, :], baseh_ref[0:1, :],
        hw_ref[0:1, :], maxi_ref[0:1, :])
    idx_ref[...] = idx
    w_ref[...] = w


def _compute_idxw(qpad, rp_pad, offs, wsi, wsj, bsi, bsj, wa, ba, hw_consts,
                  base_consts, block_rows):
    npad = qpad.shape[0]
    nb = offs.shape[0]
    offs2 = jnp.broadcast_to(
        jnp.pad(offs, (0, 128 - nb)).reshape(1, 128), (8, 128))
    bsi2 = jnp.broadcast_to(bsi.reshape(1, 128), (8, 128))
    bsj2 = jnp.broadcast_to(bsj.reshape(1, 128), (8, 128))
    ba2 = jnp.broadcast_to(ba.reshape(1, 128), (8, 128))
    col = jnp.arange(128, dtype=jnp.int32)
    gmat = (col[:, None] // 16 == col[None, :] // 16).astype(jnp.float32)
    level = (col % 16) // 4
    hw_v = jnp.array(hw_consts, jnp.int32)[level]
    base_v = jnp.array(base_consts, jnp.int32)[level] + col // 16
    scale_v = hw_v.astype(jnp.float32) * (1.0 / float(hw_consts[3]))
    maxi_v = hw_v - 1
    scale2 = jnp.broadcast_to(scale_v.reshape(1, 128), (8, 128))
    baseh2 = jnp.broadcast_to(base_v.reshape(1, 128), (8, 128))
    hw2 = jnp.broadcast_to(hw_v.reshape(1, 128), (8, 128))
    maxi2 = jnp.broadcast_to(maxi_v.reshape(1, 128), (8, 128))
    cspec = pl.BlockSpec((8, 128), lambda i: (0, 0))
    return pl.pallas_call(
        functools.partial(_idxw_kernel, nb),
        grid=(npad // block_rows,),
        in_specs=[
            pl.BlockSpec((block_rows, _EMBED), lambda i: (i, 0)),
            pl.BlockSpec((block_rows, 2), lambda i: (i, 0)),
            cspec,
            pl.BlockSpec((_EMBED, 128), lambda i: (0, 0)),
            pl.BlockSpec((_EMBED, 128), lambda i: (0, 0)),
            cspec,
            cspec,
            pl.BlockSpec((_EMBED, 128), lambda i: (0, 0)),
            cspec,
            pl.BlockSpec((128, 128), lambda i: (0, 0)),
            cspec,
            cspec,
            cspec,
            cspec,
        ],
        out_specs=[
            pl.BlockSpec((block_rows, 512), lambda i: (i, 0)),
            pl.BlockSpec((block_rows, 512), lambda i: (i, 0)),
        ],
        out_shape=[
            jax.ShapeDtypeStruct((npad, 512), jnp.int32),
            jax.ShapeDtypeStruct((npad, 512), jnp.float32),
        ],
    )(qpad, rp_pad, offs2, wsi, wsj, bsi2, bsj2, wa, ba2, gmat, scale2,
      baseh2, hw2, maxi2)


def _make_sc_gather(npad, nrows):
    per_w = npad // _NW
    nblk = per_w // _QB
    mesh = plsc.VectorSubcoreMesh(core_axis_name="c", subcore_axis_name="s")

    @functools.partial(
        pl.kernel,
        out_type=jax.ShapeDtypeStruct((npad, _EMBED), jnp.float32),
        mesh=mesh,
        scratch_types=[
            pltpu.VMEM((2, _QB, 4, 128), jnp.int32),
            pltpu.VMEM((2, _QB, 512), jnp.float32),
            pltpu.VMEM((4, 4, 128, 16), jnp.int32),
            pltpu.VMEM((_QB, _EMBED), jnp.float32),
            pltpu.SemaphoreType.DMA,
            pltpu.SemaphoreType.DMA,
        ],
        compiler_params=pltpu.CompilerParams(use_tc_tiling_on_sc=False),
    )
    def sc_kernel(table_hbm, idx_hbm, w_hbm, out_hbm, idx_v, w_v, rows_v,
                  out_v, sem, semb):
        wid = lax.axis_index("s") * 2 + lax.axis_index("c")

        def start_blk(b):
            st = wid * per_w + b * _QB
            ib = lax.rem(b, 2)
            pltpu.async_copy(idx_hbm.at[pl.ds(st, _QB)], idx_v.at[ib], semb)
            pltpu.async_copy(w_hbm.at[pl.ds(st, _QB)], w_v.at[ib], semb)

        def wait_blk(b):
            st = wid * per_w + b * _QB
            ib = lax.rem(b, 2)
            pltpu.make_async_copy(idx_hbm.at[pl.ds(st, _QB)], idx_v.at[ib],
                                  semb).wait()
            pltpu.make_async_copy(w_hbm.at[pl.ds(st, _QB)], w_v.at[ib],
                                  semb).wait()

        start_blk(0)

        def blk_body(bi, carry):
            start = wid * per_w + bi * _QB
            ib = lax.rem(bi, 2)
            wait_blk(bi)

            @pl.when(bi + 1 < nblk)
            def _():
                start_blk(bi + 1)

            def issue(qi, buf):
                for c in range(4):
                    pltpu.async_copy(table_hbm.at[idx_v.at[ib, qi, c]],
                                     rows_v.at[buf, c], sem)

            for k in range(3):
                issue(k, k)

            def q_body(qi, c2):
                buf = lax.rem(qi, 4)

                @pl.when(qi + 3 < _QB)
                def _():
                    issue(qi + 3, lax.rem(qi + 3, 4))

                for c in range(4):
                    pltpu.make_async_copy(table_hbm.at[idx_v.at[ib, qi, c]],
                                          rows_v.at[buf, c], sem).wait()
                zero = jnp.zeros((16,), jnp.float32)
                accs = (zero,) * (2 * _HEADS)

                for c in range(4):
                    wvecs = [
                        w_v[ib, qi, pl.ds(c * 128 + h * 16, 16)]
                        for h in range(_HEADS)
                    ]

                    def t_body(t, accs, c=c, wvecs=wvecs):
                        accs = list(accs)
                        tvec = jnp.full((16, 1), t, jnp.int32)
                        dnums = lax.GatherDimensionNumbers(
                            offset_dims=(), collapsed_slice_dims=(0,),
                            start_index_map=(0,))
                        for h in range(_HEADS):
                            wb = lax.gather(
                                wvecs[h], tvec, dnums, (1,),
                                mode=lax.GatherScatterMode.PROMISE_IN_BOUNDS)
                            r = h * 16 + t
                            u = rows_v[buf, c, r, :]
                            r0 = lax.bitcast_convert_type(u << 16,
                                                          jnp.float32)
                            r1 = lax.bitcast_convert_type(
                                u & jnp.int32(-65536), jnp.float32)
                            accs[2 * h] = accs[2 * h] + wb * r0
                            accs[2 * h + 1] = accs[2 * h + 1] + wb * r1
                        return tuple(accs)

                    accs = lax.fori_loop(0, 16, t_body, accs)
                for h in range(_HEADS):
                    out_v[qi, pl.ds(h * 32, 16)] = accs[2 * h]
                    out_v[qi, pl.ds(h * 32 + 16, 16)] = accs[2 * h + 1]
                return c2

            lax.fori_loop(0, _QB, q_body, 0)
            pltpu.sync_copy(out_v, out_hbm.at[pl.ds(start, _QB)])
            return carry

        lax.fori_loop(0, nblk, blk_body, 0)

    return sc_kernel


def kernel(query, query_offsets, reference_points, value_l0, value_l1,
           value_l2, value_l3, W_sampling, b_sampling, W_attn, b_attn,
           W_value, b_value, W_out, b_out):
    values = [value_l0, value_l1, value_l2, value_l3]
    N = query.shape[0]
    B = value_l0.shape[0]

    vals2d = [v.reshape(-1, _EMBED) for v in values]
    P = sum(v.shape[0] for v in vals2d)
    table = _project_values(vals2d, W_value, b_value, block_rows=512)
    table = table.reshape(P * _HEADS, 16)

    hw_consts = [v.shape[1] for v in values]
    pix_prefix = []
    acc = 0
    for v in values:
        pix_prefix.append(acc)
        acc += B * v.shape[1] * v.shape[2]
    base_consts = [p * _HEADS for p in pix_prefix]

    npad = ((N + _NW * _QB - 1) // (_NW * _QB)) * (_NW * _QB)
    qpad = jnp.pad(query, ((0, npad - N), (0, 0)))
    rp_pad = jnp.pad(reference_points, ((0, npad - N), (0, 0)))

    wsi = W_sampling[:, 0::2]
    wsj = W_sampling[:, 1::2]
    bsi = b_sampling[0::2]
    bsj = b_sampling[1::2]
    idx, w = _compute_idxw(qpad, rp_pad, query_offsets, wsi, wsj, bsi, bsj,
                           W_attn, b_attn, hw_consts, base_consts,
                           block_rows=1024)

    idx3 = idx.reshape(npad, 4, 128)
    sc_gather = _make_sc_gather(npad, table.shape[0])
    sampled = sc_gather(table, idx3, w)

    j = np.arange(_HD)
    within = np.where(j < 16, 2 * j, 2 * (j - 16) + 1)
    perm = (np.arange(_EMBED) // _HD) * _HD + within[np.arange(_EMBED) % _HD]
    w_out_perm = W_out[jnp.asarray(perm)]
    return _matmul_bias(sampled, w_out_perm, b_out, block_rows=1024,
                        out_rows=N)
